# baseline (speedup 1.0000x reference)
import math

import jax
import jax.numpy as jnp
from jax.experimental import pallas as pl
from jax.experimental.pallas import tpu as pltpu

_IN = 512
_OUT = 512
_SCALE = 1.0 / math.sqrt(_IN)
_BLOCK = 512


def _keys_body(mu_ref, sig_ref, keys_ref, mu16_ref):
    keys_ref[:] = (mu_ref[:] * jax.nn.softplus(sig_ref[:])).astype(jnp.bfloat16)
    mu16_ref[:] = mu_ref[:].astype(jnp.bfloat16)


def _main_body(x_ref, keys_ref, mu_ref, gate_ref, bias_ref, scores_ref, out_ref):
    xb = x_ref[:].astype(jnp.bfloat16)
    scores = jax.lax.dot_general(
        xb, keys_ref[:], (((1,), (1,)), ((), ())),
        preferred_element_type=jnp.float32) * _SCALE
    comp = jax.lax.dot_general(
        xb, mu_ref[:], (((1,), (1,)), ((), ())),
        preferred_element_type=jnp.float32)
    w = jnp.maximum(scores - gate_ref[:], 0.0)
    scores_ref[:] = scores
    out_ref[:] = comp * w + bias_ref[:]


def kernel(x, mu_weight, sigma_weight, gate_param, mu_bias):
    x2 = x.reshape(-1, _IN)
    tokens = x2.shape[0]

    keys, mu16 = pl.pallas_call(
        _keys_body,
        out_shape=[
            jax.ShapeDtypeStruct((_OUT, _IN), jnp.bfloat16),
            jax.ShapeDtypeStruct((_OUT, _IN), jnp.bfloat16),
        ],
    )(mu_weight, sigma_weight)

    gate2 = gate_param.reshape(1, _OUT)
    bias2 = mu_bias.reshape(1, _OUT)

    scores, masked = pl.pallas_call(
        _main_body,
        grid=(tokens // _BLOCK,),
        in_specs=[
            pl.BlockSpec((_BLOCK, _IN), lambda i: (i, 0)),
            pl.BlockSpec((_OUT, _IN), lambda i: (0, 0)),
            pl.BlockSpec((_OUT, _IN), lambda i: (0, 0)),
            pl.BlockSpec((1, _OUT), lambda i: (0, 0)),
            pl.BlockSpec((1, _OUT), lambda i: (0, 0)),
        ],
        out_specs=[
            pl.BlockSpec((_BLOCK, _OUT), lambda i: (i, 0)),
            pl.BlockSpec((_BLOCK, _OUT), lambda i: (i, 0)),
        ],
        out_shape=[
            jax.ShapeDtypeStruct((tokens, _OUT), jnp.float32),
            jax.ShapeDtypeStruct((tokens, _OUT), jnp.float32),
        ],
        compiler_params=pltpu.CompilerParams(
            dimension_semantics=("parallel",),
        ),
    )(x2, keys, mu16, gate2, bias2)

    final = masked.reshape(*x.shape[:-1], _OUT)
    return (final, scores, masked)


# speedup vs baseline: 1.2183x; 1.2183x over previous
import math

import jax
import jax.numpy as jnp
from jax.experimental import pallas as pl
from jax.experimental.pallas import tpu as pltpu

_IN = 512
_OUT = 512
_SCALE = 1.0 / math.sqrt(_IN)
_BLOCK = 2048


def _keys_body(mu_ref, sig_ref, keys_ref, mu16_ref):
    keys_ref[:] = (mu_ref[:] * jax.nn.softplus(sig_ref[:])).astype(jnp.bfloat16)
    mu16_ref[:] = mu_ref[:].astype(jnp.bfloat16)


def _main_body(x_ref, keys_ref, mu_ref, gate_ref, bias_ref, scores_ref, out_ref):
    xb = x_ref[:].astype(jnp.bfloat16)
    scores = jax.lax.dot_general(
        xb, keys_ref[:], (((1,), (1,)), ((), ())),
        preferred_element_type=jnp.float32) * _SCALE
    comp = jax.lax.dot_general(
        xb, mu_ref[:], (((1,), (1,)), ((), ())),
        preferred_element_type=jnp.float32)
    w = jnp.maximum(scores - gate_ref[:], 0.0)
    scores_ref[:] = scores
    out_ref[:] = comp * w + bias_ref[:]


def kernel(x, mu_weight, sigma_weight, gate_param, mu_bias):
    x2 = x.reshape(-1, _IN)
    tokens = x2.shape[0]

    keys, mu16 = pl.pallas_call(
        _keys_body,
        out_shape=[
            jax.ShapeDtypeStruct((_OUT, _IN), jnp.bfloat16),
            jax.ShapeDtypeStruct((_OUT, _IN), jnp.bfloat16),
        ],
    )(mu_weight, sigma_weight)

    gate2 = gate_param.reshape(1, _OUT)
    bias2 = mu_bias.reshape(1, _OUT)

    scores, masked = pl.pallas_call(
        _main_body,
        grid=(tokens // _BLOCK,),
        in_specs=[
            pl.BlockSpec((_BLOCK, _IN), lambda i: (i, 0)),
            pl.BlockSpec((_OUT, _IN), lambda i: (0, 0)),
            pl.BlockSpec((_OUT, _IN), lambda i: (0, 0)),
            pl.BlockSpec((1, _OUT), lambda i: (0, 0)),
            pl.BlockSpec((1, _OUT), lambda i: (0, 0)),
        ],
        out_specs=[
            pl.BlockSpec((_BLOCK, _OUT), lambda i: (i, 0)),
            pl.BlockSpec((_BLOCK, _OUT), lambda i: (i, 0)),
        ],
        out_shape=[
            jax.ShapeDtypeStruct((tokens, _OUT), jnp.float32),
            jax.ShapeDtypeStruct((tokens, _OUT), jnp.float32),
        ],
        compiler_params=pltpu.CompilerParams(
            dimension_semantics=("parallel",),
        ),
    )(x2, keys, mu16, gate2, bias2)

    final = masked.reshape(*x.shape[:-1], _OUT)
    return (final, scores, masked)


# speedup vs baseline: 1.3007x; 1.0676x over previous
import math

import jax
import jax.numpy as jnp
from jax.experimental import pallas as pl
from jax.experimental.pallas import tpu as pltpu

_IN = 512
_OUT = 512
_SCALE = 1.0 / math.sqrt(_IN)
_BLOCK = 2048


def _main_body(x_ref, mu_ref, sig_ref, gate_ref, bias_ref, scores_ref, out_ref):
    xb = x_ref[:].astype(jnp.bfloat16)
    mu = mu_ref[:]
    keys = (mu * jax.nn.softplus(sig_ref[:])).astype(jnp.bfloat16)
    scores = jax.lax.dot_general(
        xb, keys, (((1,), (1,)), ((), ())),
        preferred_element_type=jnp.float32) * _SCALE
    comp = jax.lax.dot_general(
        xb, mu.astype(jnp.bfloat16), (((1,), (1,)), ((), ())),
        preferred_element_type=jnp.float32)
    w = jnp.maximum(scores - gate_ref[:], 0.0)
    scores_ref[:] = scores
    out_ref[:] = comp * w + bias_ref[:]


def kernel(x, mu_weight, sigma_weight, gate_param, mu_bias):
    x2 = x.reshape(-1, _IN)
    tokens = x2.shape[0]

    gate2 = gate_param.reshape(1, _OUT)
    bias2 = mu_bias.reshape(1, _OUT)

    scores, masked = pl.pallas_call(
        _main_body,
        grid=(tokens // _BLOCK,),
        in_specs=[
            pl.BlockSpec((_BLOCK, _IN), lambda i: (i, 0)),
            pl.BlockSpec((_OUT, _IN), lambda i: (0, 0)),
            pl.BlockSpec((_OUT, _IN), lambda i: (0, 0)),
            pl.BlockSpec((1, _OUT), lambda i: (0, 0)),
            pl.BlockSpec((1, _OUT), lambda i: (0, 0)),
        ],
        out_specs=[
            pl.BlockSpec((_BLOCK, _OUT), lambda i: (i, 0)),
            pl.BlockSpec((_BLOCK, _OUT), lambda i: (i, 0)),
        ],
        out_shape=[
            jax.ShapeDtypeStruct((tokens, _OUT), jnp.float32),
            jax.ShapeDtypeStruct((tokens, _OUT), jnp.float32),
        ],
        compiler_params=pltpu.CompilerParams(
            dimension_semantics=("parallel",),
        ),
    )(x2, mu_weight, sigma_weight, gate2, bias2)

    final = masked.reshape(*x.shape[:-1], _OUT)
    return (final, scores, masked)


# speedup vs baseline: 1.3059x; 1.0040x over previous
import math

import jax
import jax.numpy as jnp
from jax.experimental import pallas as pl
from jax.experimental.pallas import tpu as pltpu

_IN = 512
_OUT = 512
_SCALE = 1.0 / math.sqrt(_IN)
_BLOCK = 2048


def _main_body(x_ref, mu_ref, sig_ref, gate_ref, bias_ref, scores_ref, out_ref):
    xb = x_ref[:].astype(jnp.bfloat16)
    mu = mu_ref[:]
    keys = (mu * jax.nn.softplus(sig_ref[:])).astype(jnp.bfloat16)
    scores = jax.lax.dot_general(
        xb, keys, (((1,), (1,)), ((), ())),
        preferred_element_type=jnp.float32) * _SCALE
    comp = jax.lax.dot_general(
        xb, mu.astype(jnp.bfloat16), (((1,), (1,)), ((), ())),
        preferred_element_type=jnp.float32)
    w = jnp.maximum(scores - gate_ref[:], 0.0)
    scores_ref[:] = scores
    out_ref[:] = comp * w + bias_ref[:]


def kernel(x, mu_weight, sigma_weight, gate_param, mu_bias):
    x2 = x.reshape(-1, _IN)
    tokens = x2.shape[0]

    gate2 = gate_param.reshape(1, _OUT)
    bias2 = mu_bias.reshape(1, _OUT)

    scores, masked = pl.pallas_call(
        _main_body,
        grid=(tokens // _BLOCK,),
        in_specs=[
            pl.BlockSpec((_BLOCK, _IN), lambda i: (i, 0)),
            pl.BlockSpec((_OUT, _IN), lambda i: (0, 0)),
            pl.BlockSpec((_OUT, _IN), lambda i: (0, 0)),
            pl.BlockSpec((1, _OUT), lambda i: (0, 0)),
            pl.BlockSpec((1, _OUT), lambda i: (0, 0)),
        ],
        out_specs=[
            pl.BlockSpec((_BLOCK, _OUT), lambda i: (i, 0)),
            pl.BlockSpec((_BLOCK, _OUT), lambda i: (i, 0)),
        ],
        out_shape=[
            jax.ShapeDtypeStruct((tokens, _OUT), jnp.float32),
            jax.ShapeDtypeStruct((tokens, _OUT), jnp.float32),
        ],
        compiler_params=pltpu.CompilerParams(
            dimension_semantics=("arbitrary",),
        ),
    )(x2, mu_weight, sigma_weight, gate2, bias2)

    final = masked.reshape(*x.shape[:-1], _OUT)
    return (final, scores, masked)
